# baseline (speedup 1.0000x reference)
import functools

import jax
import jax.numpy as jnp
from jax import lax
from jax.experimental import pallas as pl
from jax.experimental.pallas import tpu as pltpu
from jax.experimental.pallas import tpu_sc as plsc

N = 10000
E = 320000
D = 128
H = 128
ED = 16
NSUB = 16
E2 = E // 2
EPT = E2 // NSUB
C = 96
NFULL = EPT // C
REM = EPT - NFULL * C
RPT = 632
NPAD = RPT * NSUB
HC = 64
HC2 = 2 * HC
TRASH = N


def _eproj_body(attr_ref, w_ref, b_ref, out_ref):
    a = attr_ref[...]
    w = w_ref[...]
    acc = lax.dot_general(a, w, (((1,), (1,)), ((), ())),
                          preferred_element_type=jnp.float32)
    out_ref[...] = acc + b_ref[0]


def _eproj(edge_attr, W_edge, b_edge2):
    BE = 4000
    return pl.pallas_call(
        _eproj_body,
        grid=(E2 // BE,),
        in_specs=[
            pl.BlockSpec((BE, ED), lambda i: (i, 0)),
            pl.BlockSpec((D, ED), lambda i: (0, 0)),
            pl.BlockSpec((1, D), lambda i: (0, 0)),
        ],
        out_specs=pl.BlockSpec((BE, D), lambda i: (i, 0)),
        out_shape=jax.ShapeDtypeStruct((E2, D), jnp.float32),
    )(edge_attr, W_edge, b_edge2)


def _msg_body(eoff, x2, src, dst, e3, in_n, in_d, out_n, out_d,
              gidx, didx, sidx,
              xb, eb, wb, nb, acc_n, acc_d,
              sem_i0, sem_i1, sem_g0, sem_g1, sem_s0, sem_s1):
    c = lax.axis_index("c")
    s = lax.axis_index("s")
    start = s * EPT
    sem_i = (sem_i0, sem_i1)
    sem_g = (sem_g0, sem_g1)
    sem_s = (sem_s0, sem_s1)
    zero16 = jnp.zeros((16,), jnp.float32)

    r0 = s * RPT
    pltpu.sync_copy(in_n.at[c, pl.ds(r0, RPT)], acc_n.at[pl.ds(r0, RPT)])
    pltpu.sync_copy(in_d.at[c, pl.ds(r0, RPT)], acc_d.at[pl.ds(r0, RPT)])
    plsc.subcore_barrier()

    def issue_idx(b, base):
        pltpu.async_copy(src.at[pl.ds(eoff + base, C)], gidx.at[b], sem_i[b])
        pltpu.async_copy(dst.at[pl.ds(eoff + base, C)], didx.at[b], sem_i[b])

    def wait_idx(b, base):
        pltpu.make_async_copy(src.at[pl.ds(eoff + base, C)], gidx.at[b],
                              sem_i[b]).wait()
        pltpu.make_async_copy(dst.at[pl.ds(eoff + base, C)], didx.at[b],
                              sem_i[b]).wait()

    def transform(b):
        for j in range(C // 16):
            sl = pl.ds(j * 16, 16)
            gidx[b, sl] = gidx[b, sl] * 2 + c

    ecol = pl.ds(c * HC, HC)

    def issue_gather(b, base):
        pltpu.async_copy(x2.at[gidx.at[b]], xb.at[b], sem_g[b])
        pltpu.async_copy(e3.at[pl.ds(base, C), ecol], eb.at[b], sem_g[b])

    def wait_gather(b, base):
        pltpu.make_async_copy(x2.at[gidx.at[b]], xb.at[b], sem_g[b]).wait()
        pltpu.make_async_copy(e3.at[pl.ds(base, C), ecol], eb.at[b],
                              sem_g[b]).wait()

    def compute(b):
        for j in range(C // 16):
            sl = pl.ds(j * 16, 16)
            sidx[b, sl] = didx[b, sl]
        def row2(i2, _):
            i = i2 * 2
            for di in range(2):
                for k in range(4):
                    sl = pl.ds(k * 16, 16)
                    t = xb[b, i + di, sl] + eb[b, i + di, sl]
                    m = jnp.maximum(t, 0.0)
                    w = jnp.exp(m)
                    wb[b, i + di, sl] = w
                    nb[b, i + di, sl] = m * w
            return 0
        lax.fori_loop(0, C // 2, row2, 0)

    def issue_scatter(b):
        pltpu.async_copy(wb.at[b], acc_d.at[sidx.at[b]], sem_s[b], add=True)
        pltpu.async_copy(nb.at[b], acc_n.at[sidx.at[b]], sem_s[b], add=True)

    def wait_scatter(b):
        pltpu.make_async_copy(wb.at[b], acc_d.at[sidx.at[b]], sem_s[b]).wait()
        pltpu.make_async_copy(nb.at[b], acc_n.at[sidx.at[b]], sem_s[b]).wait()

    for b in (0, 1):
        for j in range(C // 16):
            sidx[b, pl.ds(j * 16, 16)] = jnp.full((16,), TRASH, jnp.int32)
        issue_scatter(b)
    issue_idx(0, start)
    wait_idx(0, start)
    transform(0)
    issue_gather(0, start)
    issue_idx(1, start + C)

    def pair(g2, _):
        for b in (0, 1):
            g = 2 * g2 + b
            nb_ = 1 - b

            @pl.when(g <= NFULL - 2)
            def _():
                wait_idx(nb_, start + (g + 1) * C)
                transform(nb_)
                issue_gather(nb_, start + (g + 1) * C)
            wait_gather(b, start + g * C)
            wait_scatter(b)
            compute(b)
            issue_scatter(b)

            @pl.when(g <= NFULL - 3)
            def _():
                issue_idx(b, start + (g + 2) * C)
        return 0

    lax.fori_loop(0, NFULL // 2, pair, 0)
    wait_scatter(0)
    wait_scatter(1)

    base = start + NFULL * C
    pltpu.sync_copy(src.at[pl.ds(eoff + base, REM)], gidx.at[0, pl.ds(0, REM)])
    pltpu.sync_copy(dst.at[pl.ds(eoff + base, REM)], didx.at[0, pl.ds(0, REM)])
    for j in range(C // 16):
        sl = pl.ds(j * 16, 16)
        if j < REM // 16:
            gidx[0, sl] = gidx[0, sl] * 2 + c
        else:
            gidx[0, sl] = jnp.zeros((16,), jnp.int32)
            didx[0, sl] = jnp.full((16,), TRASH, jnp.int32)
    def zero_etail(i, _):
        for k in range(4):
            eb[0, i, pl.ds(k * 16, 16)] = zero16
        return 0
    lax.fori_loop(REM, C, zero_etail, 0)
    pltpu.async_copy(x2.at[gidx.at[0]], xb.at[0], sem_g[0])
    pltpu.async_copy(e3.at[pl.ds(base, REM), ecol], eb.at[0, pl.ds(0, REM)],
                     sem_g[0])
    pltpu.make_async_copy(x2.at[gidx.at[0]], xb.at[0], sem_g[0]).wait()
    pltpu.make_async_copy(e3.at[pl.ds(base, REM), ecol],
                          eb.at[0, pl.ds(0, REM)], sem_g[0]).wait()
    compute(0)
    issue_scatter(0)
    wait_scatter(0)

    plsc.subcore_barrier()

    pltpu.sync_copy(acc_n.at[pl.ds(r0, RPT)], out_n.at[c, pl.ds(r0, RPT)])
    pltpu.sync_copy(acc_d.at[pl.ds(r0, RPT)], out_d.at[c, pl.ds(r0, RPT)])


def _msgpass(x2, src, dst, e3, in_n, in_d, eoff):
    fn = functools.partial(
        pl.kernel,
        out_type=(jax.ShapeDtypeStruct((2, NPAD, HC), jnp.float32),
                  jax.ShapeDtypeStruct((2, NPAD, HC), jnp.float32)),
        mesh=plsc.VectorSubcoreMesh(core_axis_name="c", subcore_axis_name="s"),
        compiler_params=pltpu.CompilerParams(use_tc_tiling_on_sc=False),
        scratch_types=[
            pltpu.VMEM((2, C), jnp.int32),
            pltpu.VMEM((2, C), jnp.int32),
            pltpu.VMEM((2, C), jnp.int32),
            pltpu.VMEM((2, C, HC), jnp.float32),
            pltpu.VMEM((2, C, HC), jnp.float32),
            pltpu.VMEM((2, C, HC), jnp.float32),
            pltpu.VMEM((2, C, HC), jnp.float32),
            pltpu.VMEM_SHARED((NPAD, HC), jnp.float32),
            pltpu.VMEM_SHARED((NPAD, HC), jnp.float32),
            pltpu.SemaphoreType.DMA,
            pltpu.SemaphoreType.DMA,
            pltpu.SemaphoreType.DMA,
            pltpu.SemaphoreType.DMA,
            pltpu.SemaphoreType.DMA,
            pltpu.SemaphoreType.DMA,
        ],
    )(functools.partial(_msg_body, eoff))
    return fn(x2, src, dst, e3, in_n, in_d)


def _mlp_body(num_ref, den_ref, x_ref, w1_ref, b1_ref, g_ref, be_ref,
              w2_ref, b2_ref, wl_ref, bl_ref, out_ref):
    x = x_ref[...]
    o_lo = num_ref[0, :N] / (den_ref[0, :N] + 1e-16) + x[:, :HC]
    o_hi = num_ref[1, :N] / (den_ref[1, :N] + 1e-16) + x[:, HC:]
    w1 = w1_ref[...]
    h = (lax.dot_general(o_lo, w1[:, :HC], (((1,), (1,)), ((), ())),
                         preferred_element_type=jnp.float32)
         + lax.dot_general(o_hi, w1[:, HC:], (((1,), (1,)), ((), ())),
                           preferred_element_type=jnp.float32)
         + b1_ref[...])
    mean = jnp.mean(h, axis=0, keepdims=True)
    var = jnp.mean(h * h, axis=0, keepdims=True) - mean * mean
    scale = g_ref[...] * lax.rsqrt(var + 1e-5)
    h = (h - mean) * scale + be_ref[...]
    h = jnp.maximum(h, 0.0)
    h = lax.dot_general(h, w2_ref[...], (((1,), (1,)), ((), ())),
                        preferred_element_type=jnp.float32) + b2_ref[...]
    h = jnp.maximum(h, 0.0)
    out_ref[...] = lax.dot_general(h, wl_ref[...], (((1,), (1,)), ((), ())),
                                   preferred_element_type=jnp.float32) + bl_ref[...]


def _mlp(numer, denom, x, W1, b1, g, be, W2, b2, Wlin, bl):
    return pl.pallas_call(
        _mlp_body,
        out_shape=jax.ShapeDtypeStruct((N, D), jnp.float32),
    )(numer, denom, x, W1, b1, g, be, W2, b2, Wlin, bl)


def kernel(x, edge_index, edge_attr, W_edge, b_edge, W1, b1, bn_gamma,
           bn_beta, W2, b2, Wlin, blin):
    b2 = b_edge.reshape(1, D)
    e3a = _eproj(edge_attr[:E2], W_edge, b2)
    e3b = _eproj(edge_attr[E2:], W_edge, b2)
    x2 = x.reshape(2 * N, HC)
    src, dst = edge_index[0], edge_index[1]
    zacc = jnp.zeros((2, NPAD, HC), jnp.float32)
    numer, denom = _msgpass(x2, src, dst, e3a, zacc, zacc, 0)
    numer, denom = _msgpass(x2, src, dst, e3b, numer, denom, E2)
    return _mlp(numer, denom, x,
                W1, b1.reshape(1, 2 * H),
                bn_gamma.reshape(1, 2 * H), bn_beta.reshape(1, 2 * H),
                W2, b2.reshape(1, H), Wlin, blin.reshape(1, H))


# speedup vs baseline: 1.0554x; 1.0554x over previous
import functools

import jax
import jax.numpy as jnp
from jax import lax
from jax.experimental import pallas as pl
from jax.experimental.pallas import tpu as pltpu
from jax.experimental.pallas import tpu_sc as plsc

N = 10000
E = 320000
D = 128
H = 128
ED = 16
NSUB = 16
E2 = E
EPT = E2 // NSUB
C = 96
NFULL = EPT // C
REM = EPT - NFULL * C
RPT = 632
NPAD = RPT * NSUB
HC = 64
HC2 = 2 * HC
TRASH = N


def _eproj_body(attr_ref, w_ref, b_ref, out_ref):
    a = attr_ref[...]
    w = w_ref[...]
    acc = lax.dot_general(a, w, (((1,), (1,)), ((), ())),
                          preferred_element_type=jnp.float32)
    out_ref[...] = acc + b_ref[0]


def _eproj(edge_attr, W_edge, b_edge2):
    BE = 4000
    return pl.pallas_call(
        _eproj_body,
        grid=(E2 // BE,),
        in_specs=[
            pl.BlockSpec((BE, ED), lambda i: (i, 0)),
            pl.BlockSpec((D, ED), lambda i: (0, 0)),
            pl.BlockSpec((1, D), lambda i: (0, 0)),
        ],
        out_specs=pl.BlockSpec((BE, D), lambda i: (i, 0)),
        out_shape=jax.ShapeDtypeStruct((E2, D), jnp.float32),
    )(edge_attr, W_edge, b_edge2)


def _msg_body(eoff, x2, src, dst, e3, in_n, in_d, out_n, out_d,
              gidx, didx, sidx,
              xb, eb, wb, nb, acc_n, acc_d,
              sem_i0, sem_i1, sem_g0, sem_g1, sem_s0, sem_s1):
    c = lax.axis_index("c")
    s = lax.axis_index("s")
    start = s * EPT
    sem_i = (sem_i0, sem_i1)
    sem_g = (sem_g0, sem_g1)
    sem_s = (sem_s0, sem_s1)
    zero16 = jnp.zeros((16,), jnp.float32)

    r0 = s * RPT
    pltpu.sync_copy(in_n.at[c, pl.ds(r0, RPT)], acc_n.at[pl.ds(r0, RPT)])
    pltpu.sync_copy(in_d.at[c, pl.ds(r0, RPT)], acc_d.at[pl.ds(r0, RPT)])
    plsc.subcore_barrier()

    def issue_idx(b, base):
        pltpu.async_copy(src.at[pl.ds(eoff + base, C)], gidx.at[b], sem_i[b])
        pltpu.async_copy(dst.at[pl.ds(eoff + base, C)], didx.at[b], sem_i[b])

    def wait_idx(b, base):
        pltpu.make_async_copy(src.at[pl.ds(eoff + base, C)], gidx.at[b],
                              sem_i[b]).wait()
        pltpu.make_async_copy(dst.at[pl.ds(eoff + base, C)], didx.at[b],
                              sem_i[b]).wait()

    def transform(b):
        for j in range(C // 16):
            sl = pl.ds(j * 16, 16)
            gidx[b, sl] = gidx[b, sl] * 2 + c

    ecol = pl.ds(c * HC, HC)

    def issue_gather(b, base):
        pltpu.async_copy(x2.at[gidx.at[b]], xb.at[b], sem_g[b])
        pltpu.async_copy(e3.at[pl.ds(base, C), ecol], eb.at[b], sem_g[b])

    def wait_gather(b, base):
        pltpu.make_async_copy(x2.at[gidx.at[b]], xb.at[b], sem_g[b]).wait()
        pltpu.make_async_copy(e3.at[pl.ds(base, C), ecol], eb.at[b],
                              sem_g[b]).wait()

    def compute(b):
        for j in range(C // 16):
            sl = pl.ds(j * 16, 16)
            sidx[b, sl] = didx[b, sl]
        def row2(i2, _):
            i = i2 * 2
            for di in range(2):
                for k in range(4):
                    sl = pl.ds(k * 16, 16)
                    t = xb[b, i + di, sl] + eb[b, i + di, sl]
                    m = jnp.maximum(t, 0.0)
                    w = jnp.exp(m)
                    wb[b, i + di, sl] = w
                    nb[b, i + di, sl] = m * w
            return 0
        lax.fori_loop(0, C // 2, row2, 0)

    def issue_scatter(b):
        pltpu.async_copy(wb.at[b], acc_d.at[sidx.at[b]], sem_s[b], add=True)
        pltpu.async_copy(nb.at[b], acc_n.at[sidx.at[b]], sem_s[b], add=True)

    def wait_scatter(b):
        pltpu.make_async_copy(wb.at[b], acc_d.at[sidx.at[b]], sem_s[b]).wait()
        pltpu.make_async_copy(nb.at[b], acc_n.at[sidx.at[b]], sem_s[b]).wait()

    for b in (0, 1):
        for j in range(C // 16):
            sidx[b, pl.ds(j * 16, 16)] = jnp.full((16,), TRASH, jnp.int32)
        issue_scatter(b)
    issue_idx(0, start)
    wait_idx(0, start)
    transform(0)
    issue_gather(0, start)
    issue_idx(1, start + C)

    def pair(g2, _):
        for b in (0, 1):
            g = 2 * g2 + b
            nb_ = 1 - b

            @pl.when(g <= NFULL - 2)
            def _():
                wait_idx(nb_, start + (g + 1) * C)
                transform(nb_)
                issue_gather(nb_, start + (g + 1) * C)
            wait_gather(b, start + g * C)
            wait_scatter(b)
            compute(b)
            issue_scatter(b)

            @pl.when(g <= NFULL - 3)
            def _():
                issue_idx(b, start + (g + 2) * C)
        return 0

    lax.fori_loop(0, NFULL // 2, pair, 0)
    wait_scatter(0)
    wait_scatter(1)

    base = start + NFULL * C
    pltpu.sync_copy(src.at[pl.ds(eoff + base, REM)], gidx.at[0, pl.ds(0, REM)])
    pltpu.sync_copy(dst.at[pl.ds(eoff + base, REM)], didx.at[0, pl.ds(0, REM)])
    for j in range(C // 16):
        sl = pl.ds(j * 16, 16)
        if j < REM // 16:
            gidx[0, sl] = gidx[0, sl] * 2 + c
        else:
            gidx[0, sl] = jnp.zeros((16,), jnp.int32)
            didx[0, sl] = jnp.full((16,), TRASH, jnp.int32)
    def zero_etail(i, _):
        for k in range(4):
            eb[0, i, pl.ds(k * 16, 16)] = zero16
        return 0
    lax.fori_loop(REM, C, zero_etail, 0)
    pltpu.async_copy(x2.at[gidx.at[0]], xb.at[0], sem_g[0])
    pltpu.async_copy(e3.at[pl.ds(base, REM), ecol], eb.at[0, pl.ds(0, REM)],
                     sem_g[0])
    pltpu.make_async_copy(x2.at[gidx.at[0]], xb.at[0], sem_g[0]).wait()
    pltpu.make_async_copy(e3.at[pl.ds(base, REM), ecol],
                          eb.at[0, pl.ds(0, REM)], sem_g[0]).wait()
    compute(0)
    issue_scatter(0)
    wait_scatter(0)

    plsc.subcore_barrier()

    pltpu.sync_copy(acc_n.at[pl.ds(r0, RPT)], out_n.at[c, pl.ds(r0, RPT)])
    pltpu.sync_copy(acc_d.at[pl.ds(r0, RPT)], out_d.at[c, pl.ds(r0, RPT)])


def _msgpass(x2, src, dst, e3, in_n, in_d, eoff):
    fn = functools.partial(
        pl.kernel,
        out_type=(jax.ShapeDtypeStruct((2, NPAD, HC), jnp.float32),
                  jax.ShapeDtypeStruct((2, NPAD, HC), jnp.float32)),
        mesh=plsc.VectorSubcoreMesh(core_axis_name="c", subcore_axis_name="s"),
        compiler_params=pltpu.CompilerParams(use_tc_tiling_on_sc=False),
        scratch_types=[
            pltpu.VMEM((2, C), jnp.int32),
            pltpu.VMEM((2, C), jnp.int32),
            pltpu.VMEM((2, C), jnp.int32),
            pltpu.VMEM((2, C, HC), jnp.float32),
            pltpu.VMEM((2, C, HC), jnp.float32),
            pltpu.VMEM((2, C, HC), jnp.float32),
            pltpu.VMEM((2, C, HC), jnp.float32),
            pltpu.VMEM_SHARED((NPAD, HC), jnp.float32),
            pltpu.VMEM_SHARED((NPAD, HC), jnp.float32),
            pltpu.SemaphoreType.DMA,
            pltpu.SemaphoreType.DMA,
            pltpu.SemaphoreType.DMA,
            pltpu.SemaphoreType.DMA,
            pltpu.SemaphoreType.DMA,
            pltpu.SemaphoreType.DMA,
        ],
    )(functools.partial(_msg_body, eoff))
    return fn(x2, src, dst, e3, in_n, in_d)


def _mlp_body(num_ref, den_ref, x_ref, w1_ref, b1_ref, g_ref, be_ref,
              w2_ref, b2_ref, wl_ref, bl_ref, out_ref):
    x = x_ref[...]
    o_lo = num_ref[0, :N] / (den_ref[0, :N] + 1e-16) + x[:, :HC]
    o_hi = num_ref[1, :N] / (den_ref[1, :N] + 1e-16) + x[:, HC:]
    w1 = w1_ref[...]
    h = (lax.dot_general(o_lo, w1[:, :HC], (((1,), (1,)), ((), ())),
                         preferred_element_type=jnp.float32)
         + lax.dot_general(o_hi, w1[:, HC:], (((1,), (1,)), ((), ())),
                           preferred_element_type=jnp.float32)
         + b1_ref[...])
    mean = jnp.mean(h, axis=0, keepdims=True)
    var = jnp.mean(h * h, axis=0, keepdims=True) - mean * mean
    scale = g_ref[...] * lax.rsqrt(var + 1e-5)
    h = (h - mean) * scale + be_ref[...]
    h = jnp.maximum(h, 0.0)
    h = lax.dot_general(h, w2_ref[...], (((1,), (1,)), ((), ())),
                        preferred_element_type=jnp.float32) + b2_ref[...]
    h = jnp.maximum(h, 0.0)
    out_ref[...] = lax.dot_general(h, wl_ref[...], (((1,), (1,)), ((), ())),
                                   preferred_element_type=jnp.float32) + bl_ref[...]


def _mlp(numer, denom, x, W1, b1, g, be, W2, b2, Wlin, bl):
    return pl.pallas_call(
        _mlp_body,
        out_shape=jax.ShapeDtypeStruct((N, D), jnp.float32),
    )(numer, denom, x, W1, b1, g, be, W2, b2, Wlin, bl)


def kernel(x, edge_index, edge_attr, W_edge, b_edge, W1, b1, bn_gamma,
           bn_beta, W2, b2, Wlin, blin):
    e3 = _eproj(edge_attr, W_edge, b_edge.reshape(1, D))
    x2 = x.reshape(2 * N, HC)
    zacc = jnp.zeros((2, NPAD, HC), jnp.float32)
    numer, denom = _msgpass(x2, edge_index[0], edge_index[1], e3, zacc, zacc, 0)
    return _mlp(numer, denom, x,
                W1, b1.reshape(1, 2 * H),
                bn_gamma.reshape(1, 2 * H), bn_beta.reshape(1, 2 * H),
                W2, b2.reshape(1, H), Wlin, blin.reshape(1, H))


# speedup vs baseline: 1.0869x; 1.0298x over previous
import functools

import jax
import jax.numpy as jnp
from jax import lax
from jax.experimental import pallas as pl
from jax.experimental.pallas import tpu as pltpu
from jax.experimental.pallas import tpu_sc as plsc

N = 10000
E = 320000
D = 128
H = 128
ED = 16
NSUB = 16
E2 = E
EPT = E2 // NSUB
C = 96
NFULL = EPT // C
REM = EPT - NFULL * C
RPT = 632
NPAD = RPT * NSUB
HC = 64
HC2 = 2 * HC
TRASH = N


def _eproj_body(attr_ref, w_ref, b_ref, out_ref):
    a = attr_ref[...]
    w = w_ref[...]
    acc = lax.dot_general(a, w, (((1,), (1,)), ((), ())),
                          preferred_element_type=jnp.float32)
    out_ref[...] = acc + b_ref[0]


def _eproj(edge_attr, W_edge, b_edge2):
    BE = 16000
    return pl.pallas_call(
        _eproj_body,
        grid=(E2 // BE,),
        in_specs=[
            pl.BlockSpec((BE, ED), lambda i: (i, 0)),
            pl.BlockSpec((D, ED), lambda i: (0, 0)),
            pl.BlockSpec((1, D), lambda i: (0, 0)),
        ],
        out_specs=pl.BlockSpec((BE, D), lambda i: (i, 0)),
        out_shape=jax.ShapeDtypeStruct((E2, D), jnp.float32),
    )(edge_attr, W_edge, b_edge2)


def _msg_body(eoff, x2, src, dst, e3, in_n, in_d, out_n, out_d,
              gidx, didx, sidx,
              xb, eb, wb, nb, acc_n, acc_d,
              sem_i0, sem_i1, sem_g0, sem_g1, sem_s0, sem_s1):
    c = lax.axis_index("c")
    s = lax.axis_index("s")
    start = s * EPT
    sem_i = (sem_i0, sem_i1)
    sem_g = (sem_g0, sem_g1)
    sem_s = (sem_s0, sem_s1)
    zero16 = jnp.zeros((16,), jnp.float32)

    r0 = s * RPT
    pltpu.sync_copy(in_n.at[c, pl.ds(r0, RPT)], acc_n.at[pl.ds(r0, RPT)])
    pltpu.sync_copy(in_d.at[c, pl.ds(r0, RPT)], acc_d.at[pl.ds(r0, RPT)])
    plsc.subcore_barrier()

    def issue_idx(b, base):
        pltpu.async_copy(src.at[pl.ds(eoff + base, C)], gidx.at[b], sem_i[b])
        pltpu.async_copy(dst.at[pl.ds(eoff + base, C)], didx.at[b], sem_i[b])

    def wait_idx(b, base):
        pltpu.make_async_copy(src.at[pl.ds(eoff + base, C)], gidx.at[b],
                              sem_i[b]).wait()
        pltpu.make_async_copy(dst.at[pl.ds(eoff + base, C)], didx.at[b],
                              sem_i[b]).wait()

    def transform(b):
        for j in range(C // 16):
            sl = pl.ds(j * 16, 16)
            gidx[b, sl] = gidx[b, sl] * 2 + c

    ecol = pl.ds(c * HC, HC)

    def issue_gather(b, base):
        pltpu.async_copy(x2.at[gidx.at[b]], xb.at[b], sem_g[b])
        pltpu.async_copy(e3.at[pl.ds(base, C), ecol], eb.at[b], sem_g[b])

    def wait_gather(b, base):
        pltpu.make_async_copy(x2.at[gidx.at[b]], xb.at[b], sem_g[b]).wait()
        pltpu.make_async_copy(e3.at[pl.ds(base, C), ecol], eb.at[b],
                              sem_g[b]).wait()

    def compute(b):
        for j in range(C // 16):
            sl = pl.ds(j * 16, 16)
            sidx[b, sl] = didx[b, sl]
        def row2(i2, _):
            i = i2 * 2
            for di in range(2):
                for k in range(4):
                    sl = pl.ds(k * 16, 16)
                    t = xb[b, i + di, sl] + eb[b, i + di, sl]
                    m = jnp.maximum(t, 0.0)
                    w = jnp.exp(m)
                    wb[b, i + di, sl] = w
                    nb[b, i + di, sl] = m * w
            return 0
        lax.fori_loop(0, C // 2, row2, 0)

    def issue_scatter(b):
        pltpu.async_copy(wb.at[b], acc_d.at[sidx.at[b]], sem_s[b], add=True)
        pltpu.async_copy(nb.at[b], acc_n.at[sidx.at[b]], sem_s[b], add=True)

    def wait_scatter(b):
        pltpu.make_async_copy(wb.at[b], acc_d.at[sidx.at[b]], sem_s[b]).wait()
        pltpu.make_async_copy(nb.at[b], acc_n.at[sidx.at[b]], sem_s[b]).wait()

    for b in (0, 1):
        for j in range(C // 16):
            sidx[b, pl.ds(j * 16, 16)] = jnp.full((16,), TRASH, jnp.int32)
        issue_scatter(b)
    issue_idx(0, start)
    wait_idx(0, start)
    transform(0)
    issue_gather(0, start)
    issue_idx(1, start + C)

    def pair(g2, _):
        for b in (0, 1):
            g = 2 * g2 + b
            nb_ = 1 - b

            @pl.when(g <= NFULL - 2)
            def _():
                wait_idx(nb_, start + (g + 1) * C)
                transform(nb_)
                issue_gather(nb_, start + (g + 1) * C)
            wait_gather(b, start + g * C)
            wait_scatter(b)
            compute(b)
            issue_scatter(b)

            @pl.when(g <= NFULL - 3)
            def _():
                issue_idx(b, start + (g + 2) * C)
        return 0

    lax.fori_loop(0, NFULL // 2, pair, 0)
    wait_scatter(0)
    wait_scatter(1)

    base = start + NFULL * C
    pltpu.sync_copy(src.at[pl.ds(eoff + base, REM)], gidx.at[0, pl.ds(0, REM)])
    pltpu.sync_copy(dst.at[pl.ds(eoff + base, REM)], didx.at[0, pl.ds(0, REM)])
    for j in range(C // 16):
        sl = pl.ds(j * 16, 16)
        if j < REM // 16:
            gidx[0, sl] = gidx[0, sl] * 2 + c
        else:
            gidx[0, sl] = jnp.zeros((16,), jnp.int32)
            didx[0, sl] = jnp.full((16,), TRASH, jnp.int32)
    def zero_etail(i, _):
        for k in range(4):
            eb[0, i, pl.ds(k * 16, 16)] = zero16
        return 0
    lax.fori_loop(REM, C, zero_etail, 0)
    pltpu.async_copy(x2.at[gidx.at[0]], xb.at[0], sem_g[0])
    pltpu.async_copy(e3.at[pl.ds(base, REM), ecol], eb.at[0, pl.ds(0, REM)],
                     sem_g[0])
    pltpu.make_async_copy(x2.at[gidx.at[0]], xb.at[0], sem_g[0]).wait()
    pltpu.make_async_copy(e3.at[pl.ds(base, REM), ecol],
                          eb.at[0, pl.ds(0, REM)], sem_g[0]).wait()
    compute(0)
    issue_scatter(0)
    wait_scatter(0)

    plsc.subcore_barrier()

    pltpu.sync_copy(acc_n.at[pl.ds(r0, RPT)], out_n.at[c, pl.ds(r0, RPT)])
    pltpu.sync_copy(acc_d.at[pl.ds(r0, RPT)], out_d.at[c, pl.ds(r0, RPT)])


def _msgpass(x2, src, dst, e3, in_n, in_d, eoff):
    fn = functools.partial(
        pl.kernel,
        out_type=(jax.ShapeDtypeStruct((2, NPAD, HC), jnp.float32),
                  jax.ShapeDtypeStruct((2, NPAD, HC), jnp.float32)),
        mesh=plsc.VectorSubcoreMesh(core_axis_name="c", subcore_axis_name="s"),
        compiler_params=pltpu.CompilerParams(use_tc_tiling_on_sc=False),
        scratch_types=[
            pltpu.VMEM((2, C), jnp.int32),
            pltpu.VMEM((2, C), jnp.int32),
            pltpu.VMEM((2, C), jnp.int32),
            pltpu.VMEM((2, C, HC), jnp.float32),
            pltpu.VMEM((2, C, HC), jnp.float32),
            pltpu.VMEM((2, C, HC), jnp.float32),
            pltpu.VMEM((2, C, HC), jnp.float32),
            pltpu.VMEM_SHARED((NPAD, HC), jnp.float32),
            pltpu.VMEM_SHARED((NPAD, HC), jnp.float32),
            pltpu.SemaphoreType.DMA,
            pltpu.SemaphoreType.DMA,
            pltpu.SemaphoreType.DMA,
            pltpu.SemaphoreType.DMA,
            pltpu.SemaphoreType.DMA,
            pltpu.SemaphoreType.DMA,
        ],
    )(functools.partial(_msg_body, eoff))
    return fn(x2, src, dst, e3, in_n, in_d)


def _mlp_body(num_ref, den_ref, x_ref, w1_ref, b1_ref, g_ref, be_ref,
              w2_ref, b2_ref, wl_ref, bl_ref, out_ref):
    x = x_ref[...]
    o_lo = num_ref[0, :N] / (den_ref[0, :N] + 1e-16) + x[:, :HC]
    o_hi = num_ref[1, :N] / (den_ref[1, :N] + 1e-16) + x[:, HC:]
    w1 = w1_ref[...]
    h = (lax.dot_general(o_lo, w1[:, :HC], (((1,), (1,)), ((), ())),
                         preferred_element_type=jnp.float32)
         + lax.dot_general(o_hi, w1[:, HC:], (((1,), (1,)), ((), ())),
                           preferred_element_type=jnp.float32)
         + b1_ref[...])
    mean = jnp.mean(h, axis=0, keepdims=True)
    var = jnp.mean(h * h, axis=0, keepdims=True) - mean * mean
    scale = g_ref[...] * lax.rsqrt(var + 1e-5)
    h = (h - mean) * scale + be_ref[...]
    h = jnp.maximum(h, 0.0)
    h = lax.dot_general(h, w2_ref[...], (((1,), (1,)), ((), ())),
                        preferred_element_type=jnp.float32) + b2_ref[...]
    h = jnp.maximum(h, 0.0)
    out_ref[...] = lax.dot_general(h, wl_ref[...], (((1,), (1,)), ((), ())),
                                   preferred_element_type=jnp.float32) + bl_ref[...]


def _mlp(numer, denom, x, W1, b1, g, be, W2, b2, Wlin, bl):
    return pl.pallas_call(
        _mlp_body,
        out_shape=jax.ShapeDtypeStruct((N, D), jnp.float32),
    )(numer, denom, x, W1, b1, g, be, W2, b2, Wlin, bl)


def kernel(x, edge_index, edge_attr, W_edge, b_edge, W1, b1, bn_gamma,
           bn_beta, W2, b2, Wlin, blin):
    e3 = _eproj(edge_attr, W_edge, b_edge.reshape(1, D))
    x2 = x.reshape(2 * N, HC)
    zacc = jnp.zeros((2, NPAD, HC), jnp.float32)
    numer, denom = _msgpass(x2, edge_index[0], edge_index[1], e3, zacc, zacc, 0)
    return _mlp(numer, denom, x,
                W1, b1.reshape(1, 2 * H),
                bn_gamma.reshape(1, 2 * H), bn_beta.reshape(1, 2 * H),
                W2, b2.reshape(1, H), Wlin, blin.reshape(1, H))
